# baseline (speedup 1.0000x reference)
import functools

import jax
import jax.numpy as jnp
from jax import lax
from jax.experimental import pallas as pl
from jax.experimental.pallas import tpu as pltpu
from jax.experimental.pallas import tpu_sc as plsc

B = 4096
L1, L2, L3 = 20, 200, 26
H = 128
HW = H // 2
NHID = 256

NC, NS = 2, 16
NW = NC * NS
CH = 128

N1, N2, N3 = B * L1, B * L2, B * L3
P1, P2, P3 = N1 // NW, N2 // NW, N3 // NW


NBUF = 4
CONCAT = H * (L1 + L2 + L3)
NWCH = CONCAT // CH


def _sc_gather():
    mesh = plsc.VectorSubcoreMesh(core_axis_name="c", subcore_axis_name="s")

    @functools.partial(
        pl.kernel,
        mesh=mesh,
        out_type=(
            jax.ShapeDtypeStruct((N1, H), jnp.float32),
            jax.ShapeDtypeStruct((N2, H), jnp.float32),
            jax.ShapeDtypeStruct((N3, H), jnp.float32),
            jax.ShapeDtypeStruct((CONCAT, H), jnp.int32),
        ),
        scratch_types=[
            pltpu.VMEM((P1,), jnp.int32),
            pltpu.VMEM((P2,), jnp.int32),
            pltpu.VMEM((P3,), jnp.int32),
            pltpu.VMEM((NBUF, CH, H), jnp.float32),
            pltpu.VMEM((CH,), jnp.int32),
            pltpu.VMEM((CH, H), jnp.int32),
            pltpu.SemaphoreType.DMA((NBUF,)),
            pltpu.SemaphoreType.DMA((NBUF,)),
        ],
    )
    def k(idx1, idx2, idx3, t1, t2, t3, perm, wt2i, o1, o2, o3, wtp,
          idx1_v, idx2_v, idx3_v, rows_v, pidx_v, wrow_v, gsem, wsem):
        wid = lax.axis_index("s") * NC + lax.axis_index("c")

        wlo = (wid * NWCH) // NW
        whi = ((wid + 1) * NWCH) // NW

        def wbody(c, _):
            off = c * CH
            pltpu.sync_copy(perm.at[pl.ds(off, CH)], pidx_v)
            pltpu.async_copy(wt2i.at[pidx_v], wrow_v, gsem.at[0]).wait()
            pltpu.sync_copy(wrow_v, wtp.at[pl.ds(off, CH)])
            return 0

        lax.fori_loop(wlo, whi, wbody, 0)

        def run(idx_hbm, idx_v, table_hbm, out_hbm, per_worker):
            n = per_worker // CH
            base = wid * per_worker
            pltpu.sync_copy(idx_hbm.at[pl.ds(base, per_worker)], idx_v)

            def gth(c, b):
                return pltpu.make_async_copy(
                    table_hbm.at[idx_v.at[pl.ds(c * CH, CH)]],
                    rows_v.at[b], gsem.at[b])

            def wb(c, b):
                return pltpu.make_async_copy(
                    rows_v.at[b], out_hbm.at[pl.ds(base + c * CH, CH)],
                    wsem.at[b])

            for b in range(NBUF):
                gth(b, b).start()

            m4 = ((n - NBUF) // NBUF) * NBUF

            def body(i, _):
                for b in range(NBUF):
                    c = i * NBUF + b
                    gth(c, b).wait()
                    wb(c, b).start()
                    wb(c, b).wait()
                    gth(c + NBUF, b).start()
                return 0

            lax.fori_loop(0, m4 // NBUF, body, 0)

            for cc in range(m4, n):
                b = cc % NBUF
                gth(cc, b).wait()
                wb(cc, b).start()
                wb(cc, b).wait()
                if cc + NBUF < n:
                    gth(cc + NBUF, b).start()

        run(idx1, idx1_v, t1, o1, P1)
        run(idx2, idx2_v, t2, o2, P2)
        run(idx3, idx3_v, t3, o3, P3)

    return k


def _w_transpose(w_inter):
    def body(w_ref, out_ref):
        out_ref[...] = w_ref[...].T.astype(jnp.bfloat16)

    return pl.pallas_call(
        body,
        grid=(NWCH,),
        in_specs=[pl.BlockSpec((NHID, CH), lambda t: (0, t))],
        out_specs=pl.BlockSpec((CH, NHID), lambda t: (t, 0)),
        out_shape=jax.ShapeDtypeStruct((CONCAT, NHID), jnp.bfloat16),
    )(w_inter)


_MB = B
_KB = 256
_NK1, _NK2, _NK3 = (L1 * H) // _KB, (L2 * H) // _KB, (L3 * H) // _KB
_NK = _NK1 + _NK2 + _NK3


def _tc_body(g1, g2, g3, w, bi, wf, bf, out_ref, acc_ref):
    k = pl.program_id(0)

    @pl.when(k == 0)
    def _():
        acc_ref[...] = jnp.broadcast_to(bi[...], (_MB, NHID))

    @pl.when(k < _NK1)
    def _():
        acc_ref[...] += jnp.dot(g1[...].astype(jnp.bfloat16), w[...],
                                preferred_element_type=jnp.float32)

    @pl.when((k >= _NK1) & (k < _NK1 + _NK2))
    def _():
        acc_ref[...] += jnp.dot(g2[...].astype(jnp.bfloat16), w[...],
                                preferred_element_type=jnp.float32)

    @pl.when(k >= _NK1 + _NK2)
    def _():
        acc_ref[...] += jnp.dot(g3[...].astype(jnp.bfloat16), w[...],
                                preferred_element_type=jnp.float32)

    @pl.when(k == _NK - 1)
    def _():
        r = acc_ref[...] * wf[...]
        out_ref[...] = jnp.sum(r, axis=1, keepdims=True) + bf[0, 0]


def _tc_mlp(g1, g2, g3, w, b_inter, w_final, b_final):
    grid = (_NK,)

    def seg_spec(lo, nk):
        return pl.BlockSpec(
            (_MB, _KB),
            lambda k: (0, jnp.clip(k - lo, 0, nk - 1)),
        )

    return pl.pallas_call(
        _tc_body,
        grid=grid,
        in_specs=[
            seg_spec(0, _NK1),
            seg_spec(_NK1, _NK2),
            seg_spec(_NK1 + _NK2, _NK3),
            pl.BlockSpec((_KB, NHID), lambda k: (k, 0)),
            pl.BlockSpec((1, NHID), lambda k: (0, 0)),
            pl.BlockSpec((1, NHID), lambda k: (0, 0)),
            pl.BlockSpec(memory_space=pltpu.SMEM),
        ],
        out_specs=pl.BlockSpec((_MB, 1), lambda k: (0, 0)),
        out_shape=jax.ShapeDtypeStruct((B, 1), jnp.float32),
        scratch_shapes=[pltpu.VMEM((_MB, NHID), jnp.float32)],
    )(g1, g2, g3, w, b_inter, w_final, b_final)


def _seg_perm(lt, off):
    return (off + jnp.arange(H)[None, :] * lt
            + jnp.arange(lt)[:, None]).reshape(-1)


def kernel(input1, input2, input3, title_emb, full_emb, cat_emb,
           W_inter, b_inter, W_final, b_final):
    idx1 = input1.reshape(-1).astype(jnp.int32)
    idx2 = input2.reshape(-1).astype(jnp.int32)
    idx3 = input3.reshape(-1).astype(jnp.int32)

    wt2 = _w_transpose(W_inter)
    wt2i = lax.bitcast_convert_type(wt2.reshape(CONCAT, H, 2), jnp.int32)
    perm = jnp.concatenate([
        _seg_perm(L1, 0), _seg_perm(L2, H * L1), _seg_perm(L3, H * (L1 + L2)),
    ]).astype(jnp.int32)

    g1, g2, g3, wtp = _sc_gather()(
        idx1, idx2, idx3, title_emb, full_emb, cat_emb, perm, wt2i)
    w = lax.bitcast_convert_type(wtp, jnp.bfloat16).reshape(CONCAT, NHID)

    return _tc_mlp(
        g1.reshape(B, L1 * H), g2.reshape(B, L2 * H), g3.reshape(B, L3 * H),
        w,
        b_inter.reshape(1, NHID),
        W_final.reshape(1, NHID),
        b_final.reshape(1, 1).astype(jnp.float32),
    )
